# baseline (speedup 1.0000x reference)
import jax
import jax.numpy as jnp
from jax.experimental import pallas as pl
from jax.experimental.pallas import tpu as pltpu

_TOTAL_TOKENS = 32768
_BATCH = 16
_D_MODEL = 4096
_HALF = _BATCH // 2


def _pooler_body(lens_ref, hs_ref, out_hbm, buf, in_sems, out_sems):
    in_copies = []
    running = lens_ref[0]
    for i in range(_BATCH):
        c = pltpu.make_async_copy(
            hs_ref.at[pl.ds(running - 1, 1)], buf.at[pl.ds(i, 1)], in_sems.at[i]
        )
        c.start()
        in_copies.append(c)
        if i + 1 < _BATCH:
            running = running + lens_ref[i + 1]

    out_copies = []
    for h in range(2):
        rows = pl.ds(h * _HALF, _HALF)
        for c in in_copies[h * _HALF : (h + 1) * _HALF]:
            c.wait()
        x = buf[rows, :]
        ss = jnp.sum(x * x, axis=1, keepdims=True)
        norm = jnp.sqrt(ss)
        scale = jnp.where(norm > 1e-12, jax.lax.rsqrt(ss), 1e12)
        buf[rows, :] = x * scale
        oc = pltpu.make_async_copy(buf.at[rows], out_hbm.at[rows], out_sems.at[h])
        oc.start()
        out_copies.append(oc)
    for oc in out_copies:
        oc.wait()


def kernel(hidden_states, extend_seq_lens):
    return pl.pallas_call(
        _pooler_body,
        out_shape=jax.ShapeDtypeStruct((_BATCH, _D_MODEL), jnp.float32),
        in_specs=[
            pl.BlockSpec(memory_space=pltpu.SMEM),
            pl.BlockSpec(memory_space=pltpu.HBM),
        ],
        out_specs=pl.BlockSpec(memory_space=pltpu.HBM),
        scratch_shapes=[
            pltpu.VMEM((_BATCH, _D_MODEL), jnp.float32),
            pltpu.SemaphoreType.DMA((_BATCH,)),
            pltpu.SemaphoreType.DMA((2,)),
        ],
    )(extend_seq_lens, hidden_states)
